# Initial kernel scaffold; baseline (speedup 1.0000x reference)
import jax
import jax.numpy as jnp
from jax.experimental import pallas as pl


def kernel(x, edge_index, W, b):
    raise NotImplementedError("write your pallas kernel here")


# speedup vs baseline: 9.8006x; 9.8006x over previous
import functools

import jax
import jax.numpy as jnp
from jax import lax
from jax.experimental import pallas as pl
from jax.experimental.pallas import tpu as pltpu
from jax.experimental.pallas import tpu_sc as plsc

_LANES = 16
_NC = 2
_NS = 16
_NW = _NC * _NS
_CP = 48
_CH = 80


def _transpose_tc(w48):
    cp, n = w48.shape

    def body(in_ref, out_ref):
        out_ref[...] = in_ref[...].T

    return pl.pallas_call(
        body,
        out_shape=jax.ShapeDtypeStruct((n, cp), w48.dtype),
    )(w48)


def _sc_accumulate(ei, wt):
    cpt = ei.shape[2]
    n, cp = wt.shape
    rpt = (n // _NS) // 8 * 8
    rem = n - _NS * rpt

    mesh = plsc.VectorSubcoreMesh(core_axis_name="c", subcore_axis_name="s")

    @functools.partial(
        pl.kernel,
        mesh=mesh,
        out_type=(
            jax.ShapeDtypeStruct((_NC, 2 * n + 16, cp), jnp.float32),
            jax.ShapeDtypeStruct((_NW, _LANES), jnp.int32),
        ),
        scratch_types=[
            pltpu.VMEM_SHARED((n, cp), jnp.float32),
            pltpu.VMEM((cpt, _CH), jnp.int32),
            pltpu.VMEM((cpt, _CH), jnp.int32),
            pltpu.VMEM((_CH, cp), jnp.float32),
            pltpu.VMEM((rpt, cp), jnp.float32),
            pltpu.VMEM((_LANES,), jnp.int32),
            pltpu.SemaphoreType.DMA,
        ],
        compiler_params=pltpu.CompilerParams(use_tc_tiling_on_sc=False),
    )
    def k1(ei_ref, wt_ref, acc_ref, min_ref, acc_s, colb, rowb, msgs, zbuf,
           minv, sem):
        c = lax.axis_index("c")
        s = lax.axis_index("s")
        tid = c * _NS + s

        zero = jnp.zeros((_LANES,), jnp.float32)

        def zrow(r, carry):
            for u in range(cp // _LANES):
                zbuf[r, pl.ds(u * _LANES, _LANES)] = zero
            return carry

        lax.fori_loop(0, rpt, zrow, 0)
        pltpu.sync_copy(zbuf, acc_s.at[pl.ds(s * rpt, rpt), :])

        @pl.when(s == 0)
        def _zero_tail():
            pltpu.sync_copy(zbuf.at[pl.ds(0, rem), :],
                            acc_s.at[pl.ds(_NS * rpt, rem), :])

        pltpu.sync_copy(ei_ref.at[1, tid], colb)
        pltpu.sync_copy(ei_ref.at[0, tid], rowb)

        def mrow(j, mm):
            for u in range(_CH // _LANES):
                mm = jnp.minimum(mm, rowb[j, pl.ds(u * _LANES, _LANES)])
            return mm

        mm = lax.fori_loop(
            0, cpt, mrow,
            jnp.full((_LANES,), jnp.iinfo(jnp.int32).max, jnp.int32))
        minv[...] = mm
        pltpu.sync_copy(minv, min_ref.at[tid])

        plsc.subcore_barrier()

        def edge(j, carry):
            pltpu.async_copy(wt_ref.at[colb.at[j]], msgs, sem).wait()
            pltpu.sync_copy(msgs, acc_s.at[rowb.at[j]], add=True)
            return carry

        lax.fori_loop(0, cpt, edge, 0)

        plsc.subcore_barrier()
        pltpu.sync_copy(acc_s.at[pl.ds(s * rpt, rpt), :],
                        acc_ref.at[c, pl.ds(s * rpt, rpt), :])

        @pl.when(s == 0)
        def _publish_tail():
            pltpu.sync_copy(acc_s.at[pl.ds(_NS * rpt, rem), :],
                            acc_ref.at[c, pl.ds(_NS * rpt, rem), :])

    return k1(ei, wt)


def _sc_combine(accf, mins, b48, n):
    cp = _CP
    nrt = (n + 16) // _NW
    nww = nrt * cp

    mesh = plsc.VectorSubcoreMesh(core_axis_name="c", subcore_axis_name="s")

    @functools.partial(
        pl.kernel,
        mesh=mesh,
        out_type=jax.ShapeDtypeStruct(((n + 16) * cp,), jnp.float32),
        scratch_types=[
            pltpu.VMEM((nww,), jnp.float32),
            pltpu.VMEM((nww,), jnp.float32),
            pltpu.VMEM((nww,), jnp.float32),
            pltpu.VMEM((cp,), jnp.float32),
            pltpu.VMEM((_NW, _LANES), jnp.int32),
        ],
        compiler_params=pltpu.CompilerParams(use_tc_tiling_on_sc=False),
    )
    def k2(acc_ref, min_ref, b_ref, out_ref, a0, a1, ob, bb, mb):
        c = lax.axis_index("c")
        s = lax.axis_index("s")
        tid = c * _NS + s

        pltpu.sync_copy(min_ref, mb)
        pltpu.sync_copy(b_ref, bb)

        def mrow(i, mm):
            return jnp.minimum(mm, mb[i, :])

        mm = lax.fori_loop(
            0, _NW, mrow,
            jnp.full((_LANES,), jnp.iinfo(jnp.int32).max, jnp.int32))
        m = mm[0]
        for j in range(1, _LANES):
            m = jnp.minimum(m, mm[j])

        r0 = tid * nrt
        off = (m + r0) * cp
        pltpu.sync_copy(acc_ref.at[0, pl.ds(off, nww)], a0)
        pltpu.sync_copy(acc_ref.at[1, pl.ds(off, nww)], a1)

        nvalid = n - m

        def row(g, carry):
            base = g * cp
            valid = (r0 + g) < nvalid
            for u in range(cp // _LANES):
                o = base + u * _LANES
                v = a0[pl.ds(o, _LANES)] + a1[pl.ds(o, _LANES)]
                v = jnp.where(valid, v, jnp.zeros_like(v))
                ob[pl.ds(o, _LANES)] = v + bb[pl.ds(u * _LANES, _LANES)]
            return carry

        lax.fori_loop(0, nrt, row, 0)
        pltpu.sync_copy(ob, out_ref.at[pl.ds(r0 * cp, nww)])

    return k2(accf, mins, b48)


def kernel(x, edge_index, W, b):
    del x
    c, n = W.shape
    e = edge_index.shape[1]

    w48 = jnp.concatenate(
        [W, jnp.zeros((_CP - c, n), W.dtype)], axis=0)
    b48 = jnp.concatenate([b, jnp.zeros((_CP - c,), b.dtype)])
    ei = edge_index.reshape(2, _NW, e // (_NW * _CH), _CH)

    wt = _transpose_tc(w48)
    acc, mins = _sc_accumulate(ei, wt)
    out48 = _sc_combine(acc.reshape(_NC, -1), mins, b48, n)
    return out48.reshape(n + 16, _CP)[:n, :c]


# speedup vs baseline: 14.2392x; 1.4529x over previous
import functools

import jax
import jax.numpy as jnp
from jax import lax
from jax.experimental import pallas as pl
from jax.experimental.pallas import tpu as pltpu
from jax.experimental.pallas import tpu_sc as plsc

_LANES = 16
_NC = 2
_NS = 16
_NW = _NC * _NS
_CP = 48
_CH = 80


def _transpose_tc(w48):
    cp, n = w48.shape

    def body(in_ref, out_ref):
        out_ref[...] = in_ref[...].T

    return pl.pallas_call(
        body,
        out_shape=jax.ShapeDtypeStruct((n, cp), w48.dtype),
    )(w48)


def _sc_accumulate(ei, wt):
    cpt = ei.shape[2]
    n, cp = wt.shape
    rpt = (n // _NS) // 8 * 8
    rem = n - _NS * rpt

    mesh = plsc.VectorSubcoreMesh(core_axis_name="c", subcore_axis_name="s")

    @functools.partial(
        pl.kernel,
        mesh=mesh,
        out_type=(
            jax.ShapeDtypeStruct((_NC, 2 * n + 16, cp), jnp.float32),
            jax.ShapeDtypeStruct((_NW, _LANES), jnp.int32),
        ),
        scratch_types=[
            pltpu.VMEM_SHARED((n, cp), jnp.float32),
            pltpu.VMEM((cpt, _CH), jnp.int32),
            pltpu.VMEM((cpt, _CH), jnp.int32),
            pltpu.VMEM((_CH, cp), jnp.float32),
            pltpu.VMEM((_CH, cp), jnp.float32),
            pltpu.VMEM((rpt, cp), jnp.float32),
            pltpu.VMEM((_LANES,), jnp.int32),
            pltpu.SemaphoreType.DMA,
            pltpu.SemaphoreType.DMA,
            pltpu.SemaphoreType.DMA,
            pltpu.SemaphoreType.DMA,
        ],
        compiler_params=pltpu.CompilerParams(use_tc_tiling_on_sc=False),
    )
    def k1(ei_ref, wt_ref, acc_ref, min_ref, acc_s, colb, rowb, msgs0, msgs1,
           zbuf, minv, gsem0, gsem1, ssem0, ssem1):
        c = lax.axis_index("c")
        s = lax.axis_index("s")
        tid = c * _NS + s

        zero = jnp.zeros((_LANES,), jnp.float32)

        def zrow(r, carry):
            for u in range(cp // _LANES):
                zbuf[r, pl.ds(u * _LANES, _LANES)] = zero
            return carry

        lax.fori_loop(0, rpt, zrow, 0)
        pltpu.sync_copy(zbuf, acc_s.at[pl.ds(s * rpt, rpt), :])

        @pl.when(s == 0)
        def _zero_tail():
            pltpu.sync_copy(zbuf.at[pl.ds(0, rem), :],
                            acc_s.at[pl.ds(_NS * rpt, rem), :])

        pltpu.sync_copy(ei_ref.at[1, tid], colb)
        pltpu.sync_copy(ei_ref.at[0, tid], rowb)

        def mrow(j, mm):
            for u in range(_CH // _LANES):
                mm = jnp.minimum(mm, rowb[j, pl.ds(u * _LANES, _LANES)])
            return mm

        mm = lax.fori_loop(
            0, cpt, mrow,
            jnp.full((_LANES,), jnp.iinfo(jnp.int32).max, jnp.int32))
        minv[...] = mm
        pltpu.sync_copy(minv, min_ref.at[tid])

        plsc.subcore_barrier()

        bufs = ((msgs0, gsem0, ssem0), (msgs1, gsem1, ssem1))

        def gather(j, buf, gsem):
            return pltpu.async_copy(wt_ref.at[colb.at[j]], buf, gsem)

        gather(0, msgs0, gsem0)
        gather(1, msgs1, gsem1)

        def step(j, buf, gsem, ssem):
            pltpu.make_async_copy(wt_ref.at[colb.at[j]], buf, gsem).wait()
            pltpu.async_copy(buf, acc_s.at[rowb.at[j]], ssem, add=True).wait()

            @pl.when(j + 2 < cpt)
            def _refill():
                gather(j + 2, buf, gsem)

        def pair(j2, carry):
            for bi, (buf, gsem, ssem) in enumerate(bufs):
                step(2 * j2 + bi, buf, gsem, ssem)
            return carry

        lax.fori_loop(0, cpt // 2, pair, 0)
        if cpt % 2:
            step(cpt - 1, *bufs[(cpt - 1) % 2])

        plsc.subcore_barrier()
        pltpu.sync_copy(acc_s.at[pl.ds(s * rpt, rpt), :],
                        acc_ref.at[c, pl.ds(s * rpt, rpt), :])

        @pl.when(s == 0)
        def _publish_tail():
            pltpu.sync_copy(acc_s.at[pl.ds(_NS * rpt, rem), :],
                            acc_ref.at[c, pl.ds(_NS * rpt, rem), :])

    return k1(ei, wt)


def _sc_combine(accf, mins, b48, n):
    cp = _CP
    nrt = (n + 16) // _NW
    nww = nrt * cp

    mesh = plsc.VectorSubcoreMesh(core_axis_name="c", subcore_axis_name="s")

    @functools.partial(
        pl.kernel,
        mesh=mesh,
        out_type=jax.ShapeDtypeStruct(((n + 16) * cp,), jnp.float32),
        scratch_types=[
            pltpu.VMEM((nww,), jnp.float32),
            pltpu.VMEM((nww,), jnp.float32),
            pltpu.VMEM((nww,), jnp.float32),
            pltpu.VMEM((cp,), jnp.float32),
            pltpu.VMEM((_NW, _LANES), jnp.int32),
        ],
        compiler_params=pltpu.CompilerParams(use_tc_tiling_on_sc=False),
    )
    def k2(acc_ref, min_ref, b_ref, out_ref, a0, a1, ob, bb, mb):
        c = lax.axis_index("c")
        s = lax.axis_index("s")
        tid = c * _NS + s

        pltpu.sync_copy(min_ref, mb)
        pltpu.sync_copy(b_ref, bb)

        def mrow(i, mm):
            return jnp.minimum(mm, mb[i, :])

        mm = lax.fori_loop(
            0, _NW, mrow,
            jnp.full((_LANES,), jnp.iinfo(jnp.int32).max, jnp.int32))
        m = mm[0]
        for j in range(1, _LANES):
            m = jnp.minimum(m, mm[j])

        r0 = tid * nrt
        off = (m + r0) * cp
        pltpu.sync_copy(acc_ref.at[0, pl.ds(off, nww)], a0)
        pltpu.sync_copy(acc_ref.at[1, pl.ds(off, nww)], a1)

        nvalid = n - m

        def row(g, carry):
            base = g * cp
            valid = (r0 + g) < nvalid
            for u in range(cp // _LANES):
                o = base + u * _LANES
                v = a0[pl.ds(o, _LANES)] + a1[pl.ds(o, _LANES)]
                v = jnp.where(valid, v, jnp.zeros_like(v))
                ob[pl.ds(o, _LANES)] = v + bb[pl.ds(u * _LANES, _LANES)]
            return carry

        lax.fori_loop(0, nrt, row, 0)
        pltpu.sync_copy(ob, out_ref.at[pl.ds(r0 * cp, nww)])

    return k2(accf, mins, b48)


def kernel(x, edge_index, W, b):
    del x
    c, n = W.shape
    e = edge_index.shape[1]

    w48 = jnp.concatenate(
        [W, jnp.zeros((_CP - c, n), W.dtype)], axis=0)
    b48 = jnp.concatenate([b, jnp.zeros((_CP - c,), b.dtype)])
    ei = edge_index.reshape(2, _NW, e // (_NW * _CH), _CH)

    wt = _transpose_tc(w48)
    acc, mins = _sc_accumulate(ei, wt)
    out48 = _sc_combine(acc.reshape(_NC, -1), mins, b48, n)
    return out48.reshape(n + 16, _CP)[:n, :c]


# speedup vs baseline: 17.9130x; 1.2580x over previous
import functools

import jax
import jax.numpy as jnp
from jax import lax
from jax.experimental import pallas as pl
from jax.experimental.pallas import tpu as pltpu
from jax.experimental.pallas import tpu_sc as plsc

_LANES = 16
_NC = 2
_NS = 16
_NW = _NC * _NS
_CP = 48
_CH = 80


def _transpose_tc(w48):
    cp, n = w48.shape

    def body(in_ref, out_ref):
        out_ref[...] = in_ref[...].T

    return pl.pallas_call(
        body,
        out_shape=jax.ShapeDtypeStruct((n, cp), w48.dtype),
    )(w48)


def _sc_accumulate(ei, wt):
    cpt = ei.shape[2]
    n, cp = wt.shape
    rpt = (n // _NS) // 8 * 8
    rem = n - _NS * rpt

    mesh = plsc.VectorSubcoreMesh(core_axis_name="c", subcore_axis_name="s")

    @functools.partial(
        pl.kernel,
        mesh=mesh,
        out_type=(
            jax.ShapeDtypeStruct((_NC, 2 * n + 16, cp), jnp.float32),
            jax.ShapeDtypeStruct((_NW, _LANES), jnp.int32),
        ),
        scratch_types=[
            pltpu.VMEM_SHARED((n, cp), jnp.float32),
            pltpu.VMEM((cpt, _CH), jnp.int32),
            pltpu.VMEM((cpt, _CH), jnp.int32),
            [pltpu.VMEM((_CH, cp), jnp.float32)] * 4,
            pltpu.VMEM((rpt, cp), jnp.float32),
            pltpu.VMEM((_LANES,), jnp.int32),
            [pltpu.SemaphoreType.DMA] * 4,
            [pltpu.SemaphoreType.DMA] * 4,
        ],
        compiler_params=pltpu.CompilerParams(use_tc_tiling_on_sc=False),
    )
    def k1(ei_ref, wt_ref, acc_ref, min_ref, acc_s, colb, rowb, msgs, zbuf,
           minv, gsems, ssems):
        c = lax.axis_index("c")
        s = lax.axis_index("s")
        tid = c * _NS + s

        zero = jnp.zeros((_LANES,), jnp.float32)

        def zrow(r, carry):
            for u in range(cp // _LANES):
                zbuf[r, pl.ds(u * _LANES, _LANES)] = zero
            return carry

        lax.fori_loop(0, rpt, zrow, 0)
        pltpu.sync_copy(zbuf, acc_s.at[pl.ds(s * rpt, rpt), :])

        @pl.when(s == 0)
        def _zero_tail():
            pltpu.sync_copy(zbuf.at[pl.ds(0, rem), :],
                            acc_s.at[pl.ds(_NS * rpt, rem), :])

        pltpu.sync_copy(ei_ref.at[1, tid], colb)
        pltpu.sync_copy(ei_ref.at[0, tid], rowb)

        def mrow(j, mm):
            for u in range(_CH // _LANES):
                mm = jnp.minimum(mm, rowb[j, pl.ds(u * _LANES, _LANES)])
            return mm

        mm = lax.fori_loop(
            0, cpt, mrow,
            jnp.full((_LANES,), jnp.iinfo(jnp.int32).max, jnp.int32))
        minv[...] = mm
        pltpu.sync_copy(minv, min_ref.at[tid])

        plsc.subcore_barrier()

        nbuf = 4

        def gather(j, b):
            pltpu.async_copy(wt_ref.at[colb.at[j]], msgs[b], gsems[b])

        for b in range(nbuf):
            gather(b, b)

        def step(j, b, refill):
            pltpu.make_async_copy(
                wt_ref.at[colb.at[j]], msgs[b], gsems[b]).wait()
            pltpu.async_copy(msgs[b], acc_s.at[rowb.at[j]], ssems[b],
                             add=True)

            @pl.when(j >= nbuf)
            def _drain_prev():
                pltpu.make_async_copy(
                    msgs[b], acc_s.at[rowb.at[j]], ssems[b]).wait()

            if refill:
                @pl.when(j + nbuf < cpt)
                def _refill():
                    gather(j + nbuf, b)

        def group(g, carry):
            for b in range(nbuf):
                step(g * nbuf + b, b, refill=True)
            return carry

        lax.fori_loop(0, cpt // nbuf, group, 0)
        for j in range(cpt - cpt % nbuf, cpt):
            step(j, j % nbuf, refill=False)
        for b in range(nbuf):
            pltpu.make_async_copy(
                msgs[b], acc_s.at[rowb.at[0]], ssems[b]).wait()

        plsc.subcore_barrier()
        pltpu.sync_copy(acc_s.at[pl.ds(s * rpt, rpt), :],
                        acc_ref.at[c, pl.ds(s * rpt, rpt), :])

        @pl.when(s == 0)
        def _publish_tail():
            pltpu.sync_copy(acc_s.at[pl.ds(_NS * rpt, rem), :],
                            acc_ref.at[c, pl.ds(_NS * rpt, rem), :])

    return k1(ei, wt)


def _sc_combine(accf, mins, b48, n):
    cp = _CP
    nrt = (n + 16) // _NW
    nww = nrt * cp

    mesh = plsc.VectorSubcoreMesh(core_axis_name="c", subcore_axis_name="s")

    @functools.partial(
        pl.kernel,
        mesh=mesh,
        out_type=jax.ShapeDtypeStruct(((n + 16) * cp,), jnp.float32),
        scratch_types=[
            pltpu.VMEM((nww,), jnp.float32),
            pltpu.VMEM((nww,), jnp.float32),
            pltpu.VMEM((nww,), jnp.float32),
            pltpu.VMEM((cp,), jnp.float32),
            pltpu.VMEM((_NW, _LANES), jnp.int32),
        ],
        compiler_params=pltpu.CompilerParams(use_tc_tiling_on_sc=False),
    )
    def k2(acc_ref, min_ref, b_ref, out_ref, a0, a1, ob, bb, mb):
        c = lax.axis_index("c")
        s = lax.axis_index("s")
        tid = c * _NS + s

        pltpu.sync_copy(min_ref, mb)
        pltpu.sync_copy(b_ref, bb)

        def mrow(i, mm):
            return jnp.minimum(mm, mb[i, :])

        mm = lax.fori_loop(
            0, _NW, mrow,
            jnp.full((_LANES,), jnp.iinfo(jnp.int32).max, jnp.int32))
        m = mm[0]
        for j in range(1, _LANES):
            m = jnp.minimum(m, mm[j])

        r0 = tid * nrt
        off = (m + r0) * cp
        pltpu.sync_copy(acc_ref.at[0, pl.ds(off, nww)], a0)
        pltpu.sync_copy(acc_ref.at[1, pl.ds(off, nww)], a1)

        nvalid = n - m

        def row(g, carry):
            base = g * cp
            valid = (r0 + g) < nvalid
            for u in range(cp // _LANES):
                o = base + u * _LANES
                v = a0[pl.ds(o, _LANES)] + a1[pl.ds(o, _LANES)]
                v = jnp.where(valid, v, jnp.zeros_like(v))
                ob[pl.ds(o, _LANES)] = v + bb[pl.ds(u * _LANES, _LANES)]
            return carry

        lax.fori_loop(0, nrt, row, 0)
        pltpu.sync_copy(ob, out_ref.at[pl.ds(r0 * cp, nww)])

    return k2(accf, mins, b48)


def kernel(x, edge_index, W, b):
    del x
    c, n = W.shape
    e = edge_index.shape[1]

    w48 = jnp.concatenate(
        [W, jnp.zeros((_CP - c, n), W.dtype)], axis=0)
    b48 = jnp.concatenate([b, jnp.zeros((_CP - c,), b.dtype)])
    ei = edge_index.reshape(2, _NW, e // (_NW * _CH), _CH)

    wt = _transpose_tc(w48)
    acc, mins = _sc_accumulate(ei, wt)
    out48 = _sc_combine(acc.reshape(_NC, -1), mins, b48, n)
    return out48.reshape(n + 16, _CP)[:n, :c]
